# baseline (speedup 1.0000x reference)
import functools

import jax
import jax.numpy as jnp
from jax import lax
from jax.experimental import pallas as pl
from jax.experimental.pallas import tpu as pltpu
from jax.experimental.pallas import tpu_sc as plsc

_C, _T, _H, _W = 3, 64, 256, 256
_TS = _T // 4
_IDX = (0, 4, 8, 12, 16, 21, 25, 29, 33, 37, 42, 46, 50, 54, 58, 63)

_ROW = _H * _W
_NROWS = _C * _T
_NSLOW = _C * _TS
_NC, _NS = 2, 16
_NW = _NC * _NS
_FPW = _NROWS // _NW


def _body(x, slow, fast, sem):
    cid = lax.axis_index("c")
    sid = lax.axis_index("s")
    w = sid * _NC + cid
    fast_copy = pltpu.make_async_copy(
        x.at[pl.ds(_FPW * _ROW * w, _FPW * _ROW)],
        fast.at[pl.ds(_FPW * _ROW * w, _FPW * _ROW)],
        sem,
    )
    fast_copy.start()
    for j in range(_NSLOW):
        owner = j % _NW
        src = (j // _TS) * _T + _IDX[j % _TS]

        @pl.when(w == owner)
        def _(src=src, j=j):
            c = pltpu.make_async_copy(
                x.at[pl.ds(src * _ROW, _ROW)],
                slow.at[pl.ds(j * _ROW, _ROW)],
                sem,
            )
            c.start()
            c.wait()

    fast_copy.wait()


@functools.partial(
    pl.kernel,
    out_type=(
        jax.ShapeDtypeStruct((_NSLOW * _ROW,), jnp.float32),
        jax.ShapeDtypeStruct((_NROWS * _ROW,), jnp.float32),
    ),
    mesh=plsc.VectorSubcoreMesh(core_axis_name="c", subcore_axis_name="s"),
    scratch_types=[pltpu.SemaphoreType.DMA],
)
def _pack_pathway(x, slow, fast, sem):
    _body(x, slow, fast, sem)


def kernel(frames):
    x = frames.reshape(_NROWS * _ROW)
    slow, fast = _pack_pathway(x)
    return (
        slow.reshape(_C, _TS, _H, _W),
        fast.reshape(_C, _T, _H, _W),
    )


# speedup vs baseline: 11.2558x; 11.2558x over previous
import functools

import jax
import jax.numpy as jnp
from jax import lax
from jax.experimental import pallas as pl
from jax.experimental.pallas import tpu as pltpu
from jax.experimental.pallas import tpu_sc as plsc

_C, _T, _H, _W = 3, 64, 256, 256
_TS = _T // 4
_IDX = (0, 4, 8, 12, 16, 21, 25, 29, 33, 37, 42, 46, 50, 54, 58, 63)

_ROW = _H * _W
_NROWS = _C * _T
_NSLOW = _C * _TS
_NC, _NS = 2, 16
_NW = _NC * _NS
_FPW = _NROWS // _NW
_CHUNK = 32768
_CPR = _ROW // _CHUNK


def _stream_rows(x, out, src_elem, dst_elem, nchunks, bufs, sem_in, sem_out):
    ins = []
    outs = []
    for k in range(nchunks):
        b = bufs[k % 2]
        ins.append(pltpu.make_async_copy(
            x.at[pl.ds(src_elem + k * _CHUNK, _CHUNK)], b, sem_in))
        outs.append(pltpu.make_async_copy(
            b, out.at[pl.ds(dst_elem + k * _CHUNK, _CHUNK)], sem_out))
    for k in range(nchunks):
        if k >= 2:
            outs[k - 2].wait()
        ins[k].start()
        ins[k].wait()
        outs[k].start()
    for k in range(max(0, nchunks - 2), nchunks):
        outs[k].wait()


def _body(x, slow, fast, buf0, buf1, sem_in, sem_out):
    cid = lax.axis_index("c")
    sid = lax.axis_index("s")
    w = sid * _NC + cid
    bufs = (buf0, buf1)
    base = w * _FPW * _ROW
    _stream_rows(x, fast, base, base, _FPW * _CPR, bufs, sem_in, sem_out)
    for j in range(_NSLOW):
        owner = j % _NW
        src = (j // _TS) * _T + _IDX[j % _TS]

        @pl.when(w == owner)
        def _(src=src, j=j):
            _stream_rows(x, slow, src * _ROW, j * _ROW, _CPR, bufs,
                         sem_in, sem_out)


@functools.partial(
    pl.kernel,
    out_type=(
        jax.ShapeDtypeStruct((_NSLOW * _ROW,), jnp.float32),
        jax.ShapeDtypeStruct((_NROWS * _ROW,), jnp.float32),
    ),
    mesh=plsc.VectorSubcoreMesh(core_axis_name="c", subcore_axis_name="s"),
    scratch_types=[
        pltpu.VMEM((_CHUNK,), jnp.float32),
        pltpu.VMEM((_CHUNK,), jnp.float32),
        pltpu.SemaphoreType.DMA,
        pltpu.SemaphoreType.DMA,
    ],
)
def _pack_pathway(x, slow, fast, buf0, buf1, sem_in, sem_out):
    _body(x, slow, fast, buf0, buf1, sem_in, sem_out)


def kernel(frames):
    x = frames.reshape(_NROWS * _ROW)
    slow, fast = _pack_pathway(x)
    return (
        slow.reshape(_C, _TS, _H, _W),
        fast.reshape(_C, _T, _H, _W),
    )


# speedup vs baseline: 30.3936x; 2.7003x over previous
import functools

import jax
import jax.numpy as jnp
from jax import lax
from jax.experimental import pallas as pl
from jax.experimental.pallas import tpu as pltpu
from jax.experimental.pallas import tpu_sc as plsc

_C, _T, _H, _W = 3, 64, 256, 256
_TS = _T // 4
_IDX = (0, 4, 8, 12, 16, 21, 25, 29, 33, 37, 42, 46, 50, 54, 58, 63)

_NROWS = _C * _T
_NSLOW = _C * _TS
_NC, _NS = 2, 16
_NW = _NC * _NS
_FPW = _NROWS // _NW
_HC = 128
_CPR = _H // _HC
_NBUF = 3


def _stream_chunks(chunks, bufs, sem_in, sem_out):
    n = len(chunks)
    ins = []
    outs = []
    for k, (src, dst) in enumerate(chunks):
        b = bufs[k % _NBUF]
        ins.append(pltpu.make_async_copy(src, b, sem_in))
        outs.append(pltpu.make_async_copy(b, dst, sem_out))
    ins[0].start()
    if n > 1:
        ins[1].start()
    for k in range(n):
        ins[k].wait()
        outs[k].start()
        if k + 2 < n:
            if k + 2 >= _NBUF:
                outs[k + 2 - _NBUF].wait()
            ins[k + 2].start()
    for k in range(max(0, n - _NBUF), n):
        outs[k].wait()


def _row_chunks(src, dst, c_s, t_s, c_d, t_d):
    out = []
    for k in range(_CPR):
        h0 = k * _HC
        out.append((
            src.at[c_s, pl.ds(t_s, 1), pl.ds(h0, _HC)],
            dst.at[c_d, pl.ds(t_d, 1), pl.ds(h0, _HC)],
        ))
    return out


def _body(x, slow, fast, buf0, buf1, buf2, sem_in, sem_out):
    cid = lax.axis_index("c")
    sid = lax.axis_index("s")
    w = sid * _NC + cid
    bufs = (buf0, buf1, buf2)
    chunks = []
    for i in range(_FPW):
        r = w * _FPW + i
        chunks += _row_chunks(x, fast, r // _T, r % _T, r // _T, r % _T)
    _stream_chunks(chunks, bufs, sem_in, sem_out)
    for j in range(_NSLOW):
        owner = j % _NW
        c = j // _TS
        t_src = _IDX[j % _TS]
        t_dst = j % _TS

        @pl.when(w == owner)
        def _(c=c, t_src=t_src, t_dst=t_dst):
            _stream_chunks(_row_chunks(x, slow, c, t_src, c, t_dst),
                           bufs, sem_in, sem_out)


@functools.partial(
    pl.kernel,
    out_type=(
        jax.ShapeDtypeStruct((_C, _TS, _H, _W), jnp.float32),
        jax.ShapeDtypeStruct((_C, _T, _H, _W), jnp.float32),
    ),
    mesh=plsc.VectorSubcoreMesh(core_axis_name="c", subcore_axis_name="s"),
    scratch_types=[
        pltpu.VMEM((1, _HC, _W), jnp.float32),
        pltpu.VMEM((1, _HC, _W), jnp.float32),
        pltpu.VMEM((1, _HC, _W), jnp.float32),
        pltpu.SemaphoreType.DMA,
        pltpu.SemaphoreType.DMA,
    ],
)
def _pack_pathway(x, slow, fast, buf0, buf1, buf2, sem_in, sem_out):
    _body(x, slow, fast, buf0, buf1, buf2, sem_in, sem_out)


def kernel(frames):
    return _pack_pathway(frames)


# speedup vs baseline: 32.9754x; 1.0849x over previous
import functools

import jax
import jax.numpy as jnp
from jax import lax
from jax.experimental import pallas as pl
from jax.experimental.pallas import tpu as pltpu
from jax.experimental.pallas import tpu_sc as plsc

_C, _T, _H, _W = 3, 64, 256, 256
_TS = _T // 4

_NROWS = _C * _T
_NC, _NS = 2, 16
_NW = _NC * _NS
_FPW = _NROWS // _NW
_HC = 128
_CPR = _H // _HC
_NBUF = 3
_NCHUNK = _FPW * _CPR


def _body(x, slow, fast, bufs, sems_in, sems_out):
    cid = lax.axis_index("c")
    sid = lax.axis_index("s")
    w = sid * _NC + cid

    ins = []
    out_fast = []
    out_slow = []
    slow_flags = []
    for k in range(_NCHUNK):
        r = w * _FPW + (k // _CPR)
        c = r // _T
        t = r % _T
        h0 = (k % _CPR) * _HC
        b = k % _NBUF
        src = x.at[c, pl.ds(t, 1), pl.ds(h0, _HC)]
        ins.append(pltpu.make_async_copy(src, bufs[b], sems_in[b]))
        out_fast.append(pltpu.make_async_copy(
            bufs[b], fast.at[c, pl.ds(t, 1), pl.ds(h0, _HC)], sems_out[b]))
        kk = (5 * t + 10) // 21
        slow_flags.append((21 * kk) // 5 == t)
        out_slow.append(pltpu.make_async_copy(
            bufs[b], slow.at[c, pl.ds(kk, 1), pl.ds(h0, _HC)], sems_out[b]))

    def drain(j):
        out_fast[j].wait()

        @pl.when(slow_flags[j])
        def _():
            out_slow[j].wait()

    ins[0].start()
    ins[1].start()
    for k in range(_NCHUNK):
        ins[k].wait()
        out_fast[k].start()

        @pl.when(slow_flags[k])
        def _(k=k):
            out_slow[k].start()

        if k + 2 < _NCHUNK:
            if k - 1 >= 0:
                drain(k - 1)
            ins[k + 2].start()
    for j in range(_NCHUNK - _NBUF, _NCHUNK):
        drain(j)


@functools.partial(
    pl.kernel,
    out_type=(
        jax.ShapeDtypeStruct((_C, _TS, _H, _W), jnp.float32),
        jax.ShapeDtypeStruct((_C, _T, _H, _W), jnp.float32),
    ),
    mesh=plsc.VectorSubcoreMesh(core_axis_name="c", subcore_axis_name="s"),
    scratch_types=[
        pltpu.VMEM((1, _HC, _W), jnp.float32),
        pltpu.VMEM((1, _HC, _W), jnp.float32),
        pltpu.VMEM((1, _HC, _W), jnp.float32),
        pltpu.SemaphoreType.DMA,
        pltpu.SemaphoreType.DMA,
        pltpu.SemaphoreType.DMA,
        pltpu.SemaphoreType.DMA,
        pltpu.SemaphoreType.DMA,
        pltpu.SemaphoreType.DMA,
    ],
)
def _pack_pathway(x, slow, fast, b0, b1, b2, si0, si1, si2, so0, so1, so2):
    _body(x, slow, fast, (b0, b1, b2), (si0, si1, si2), (so0, so1, so2))


def kernel(frames):
    return _pack_pathway(frames)
